# baseline (speedup 1.0000x reference)
import functools

import jax
import jax.numpy as jnp
from jax import lax
from jax.experimental import pallas as pl
from jax.experimental.pallas import tpu as pltpu
from jax.experimental.pallas import tpu_sc as plsc

NEG_SLOPE = 0.2
EPS = 1e-16

_NC = 2
_NS = 16
_NW = _NC * _NS
_L = 16
_CHUNK = 64
_BLKC = 16


def _tc_body(x_ref, wt_ref, as_ref, ad_ref, xw_ref, asum_ref, adsum_ref):
    xw = jnp.dot(x_ref[...], wt_ref[...], preferred_element_type=jnp.float32)
    xw_ref[...] = xw
    asum_ref[...] = jnp.sum(xw * as_ref[...], axis=1, keepdims=True)
    adsum_ref[...] = jnp.sum(xw * ad_ref[...], axis=1, keepdims=True)


def _make_sc_kernel(n_src, n_dst, n_ptr_pad, n_ad_pad, c_dim, d_per):
    n_pad = _NW * d_per
    fvecs = c_dim // _L

    def body(ptr_hbm, idx_hbm, xw_hbm, as_hbm, ad_hbm, bias_hbm, out_hbm,
             ptr_v, as_v, ad_v, bias_v,
             blk_v, rows_0, rows_1, rows_2, rows_3,
             asg_v, s_v, out_v, den_v, sem_0, sem_1, sem_2, sem_3):
        cid = lax.axis_index("c")
        sid = lax.axis_index("s")
        wid = sid * _NC + cid
        d0 = wid * d_per
        d_hi = jnp.minimum(d0 + d_per, n_dst)

        pltpu.sync_copy(ptr_hbm.at[pl.ds(d0, d_per + 24)], ptr_v)
        pltpu.sync_copy(as_hbm, as_v)
        pltpu.sync_copy(ad_hbm.at[pl.ds(d0, d_per + 16)], ad_v)
        pltpu.sync_copy(bias_hbm, bias_v)

        zero16 = jnp.zeros((_L,), jnp.float32)

        def iload(ref, i):
            return ref[pl.ds(i, _L)][0]

        def zrow(r, _):
            for f in range(fvecs):
                out_v[r, pl.ds(f * _L, _L)] = zero16
            den_v[r, :] = zero16
            return 0
        lax.fori_loop(0, d_per, zrow, 0)

        e_lo = iload(ptr_v, 0)
        e_hi = iload(ptr_v, d_hi - d0)
        c_lo = e_lo // _CHUNK
        c_hi = (e_hi + _CHUNK - 1) // _CHUNK
        nchunks = c_hi - c_lo

        def ensure_block(c):
            @pl.when(c % _BLKC == 0)
            def _():
                b = c // _BLKC
                pltpu.sync_copy(idx_hbm.at[pl.ds(b * _BLKC, _BLKC)],
                                blk_v.at[b & 1])

        def fetch(c, rows_buf, sem):
            pltpu.make_async_copy(
                xw_hbm.at[blk_v.at[(c // _BLKC) & 1, c % _BLKC]], rows_buf, sem
            ).start()

        def process_run(d, ce, rows_buf):
            dloc = d - d0
            es = iload(ptr_v, dloc)
            ee = iload(ptr_v, dloc + 1)
            run_lo = jnp.maximum(es, ce)
            run_hi = jnp.minimum(ee, ce + _CHUNK)

            @pl.when(run_hi > run_lo)
            def _():
                a_d_s = iload(ad_v, dloc)
                v_lo = (run_lo - ce) // _L
                v_hi = (run_hi - ce + _L - 1) // _L

                acc0 = tuple(out_v[dloc, pl.ds(f * _L, _L)]
                             for f in range(fvecs))

                def vec_body(vi, acc):
                    base = ce + vi * _L
                    pos = base + lax.iota(jnp.int32, _L)
                    m = (pos >= run_lo) & (pos < run_hi)
                    a = asg_v[pl.ds(vi * _L, _L)] + a_d_s
                    a = jnp.where(a >= 0, a, NEG_SLOPE * a)
                    s = jnp.where(m, jnp.exp(a), 0.0)
                    plsc.addupdate(den_v.at[dloc, :], s)
                    acc = list(acc)
                    for j in range(_L):
                        sj = s[j]
                        r = vi * _L + j
                        for f in range(fvecs):
                            acc[f] = acc[f] + sj * rows_buf[r, pl.ds(f * _L, _L)]
                    return tuple(acc)
                acc = lax.fori_loop(v_lo, v_hi, vec_body, acc0)
                for f in range(fvecs):
                    out_v[dloc, pl.ds(f * _L, _L)] = acc[f]

        def process(c, d, rows_cur, sem_cur, rows_pre, sem_pre):
            @pl.when(c + 3 < c_hi)
            def _():
                ensure_block(c + 3)
                fetch(c + 3, rows_pre, sem_pre)
            pltpu.make_async_copy(
                xw_hbm.at[blk_v.at[(c // _BLKC) & 1, c % _BLKC]],
                rows_cur, sem_cur
            ).wait()
            for v in range(_CHUNK // _L):
                ivec = blk_v[(c // _BLKC) & 1, c % _BLKC, pl.ds(v * _L, _L)]
                asg_v[pl.ds(v * _L, _L)] = plsc.load_gather(as_v, [ivec])
            ce = c * _CHUNK
            def wcond(dd):
                return jnp.logical_and(dd < d_hi,
                                       iload(ptr_v, dd + 1 - d0) <= ce + _CHUNK)

            def wbody(dd):
                process_run(dd, ce, rows_cur)
                return dd + 1
            d = lax.while_loop(wcond, wbody, d)

            @pl.when(jnp.logical_and(d < d_hi, iload(ptr_v, d - d0) < ce + _CHUNK))
            def _():
                process_run(d, ce, rows_cur)
            return d

        rows_bufs = (rows_0, rows_1, rows_2, rows_3)
        sems = (sem_0, sem_1, sem_2, sem_3)

        @pl.when(nchunks > 0)
        def _():
            b0 = c_lo // _BLKC
            pltpu.sync_copy(idx_hbm.at[pl.ds(b0 * _BLKC, _BLKC)],
                            blk_v.at[b0 & 1])
            fetch(c_lo, rows_0, sem_0)
        for k in (1, 2):
            @pl.when(nchunks > k)
            def _(k=k):
                ensure_block(c_lo + k)
                fetch(c_lo + k, rows_bufs[k], sems[k])

        nquads = (nchunks + 3) // 4

        def quad_body(qq, d):
            c0 = c_lo + 4 * qq
            for k in range(4):
                def _p(dd, k=k):
                    return process(c0 + k, dd, rows_bufs[k], sems[k],
                                   rows_bufs[(k + 3) % 4], sems[(k + 3) % 4])
                d = lax.cond(c0 + k < c_hi, _p, lambda dd: dd, d)
            return d
        lax.fori_loop(0, nquads, quad_body, d0)

        def fin(r, _):
            dtot = jnp.sum(den_v[r, :])
            denv = jnp.full((_L,), dtot) + EPS
            for f in range(fvecs):
                out_v[r, pl.ds(f * _L, _L)] = (
                    out_v[r, pl.ds(f * _L, _L)] / denv
                    + bias_v[pl.ds(f * _L, _L)])
            return 0
        lax.fori_loop(0, d_per, fin, 0)
        pltpu.sync_copy(out_v, out_hbm.at[pl.ds(d0, d_per)])

    mesh = plsc.VectorSubcoreMesh(core_axis_name="c", subcore_axis_name="s",
                                  num_cores=_NC, num_subcores=_NS)
    return pl.kernel(
        body,
        out_type=jax.ShapeDtypeStruct((n_pad, c_dim), jnp.float32),
        mesh=mesh,
        compiler_params=pltpu.CompilerParams(needs_layout_passes=False),
        scratch_types=[
            pltpu.VMEM((d_per + 24,), jnp.int32),
            pltpu.VMEM((n_src,), jnp.float32),
            pltpu.VMEM((d_per + 16,), jnp.float32),
            pltpu.VMEM((c_dim,), jnp.float32),
            pltpu.VMEM((2, _BLKC, _CHUNK), jnp.int32),
            pltpu.VMEM((_CHUNK, c_dim), jnp.float32),
            pltpu.VMEM((_CHUNK, c_dim), jnp.float32),
            pltpu.VMEM((_CHUNK, c_dim), jnp.float32),
            pltpu.VMEM((_CHUNK, c_dim), jnp.float32),
            pltpu.VMEM((_CHUNK,), jnp.float32),
            pltpu.VMEM((2 * _L,), jnp.float32),
            pltpu.VMEM((d_per, c_dim), jnp.float32),
            pltpu.VMEM((d_per, _L), jnp.float32),
            pltpu.SemaphoreType.DMA,
            pltpu.SemaphoreType.DMA,
            pltpu.SemaphoreType.DMA,
            pltpu.SemaphoreType.DMA,
        ],
    )


def kernel(x, ptr, idx, num_dst, num_src, num_edge, lin_src, att_src, att_dst, bias):
    n, d_in = x.shape
    hc = lin_src.shape[0]
    n_dst = ptr.shape[0] - 1
    e = idx.shape[0]

    rb = 2000
    tc = pl.pallas_call(
        _tc_body,
        grid=(n // rb,),
        in_specs=[
            pl.BlockSpec((rb, d_in), lambda i: (i, 0)),
            pl.BlockSpec((d_in, hc), lambda i: (0, 0)),
            pl.BlockSpec((1, hc), lambda i: (0, 0)),
            pl.BlockSpec((1, hc), lambda i: (0, 0)),
        ],
        out_specs=[
            pl.BlockSpec((rb, hc), lambda i: (i, 0)),
            pl.BlockSpec((rb, 1), lambda i: (i, 0)),
            pl.BlockSpec((rb, 1), lambda i: (i, 0)),
        ],
        out_shape=[
            jax.ShapeDtypeStruct((n, hc), jnp.float32),
            jax.ShapeDtypeStruct((n, 1), jnp.float32),
            jax.ShapeDtypeStruct((n, 1), jnp.float32),
        ],
    )
    xw, a_s, a_d = tc(x, lin_src.T, att_src.reshape(1, hc), att_dst.reshape(1, hc))
    a_s = a_s.reshape(n)
    a_d = a_d.reshape(n)[:n_dst]

    d_per = ((n_dst + _NW - 1) // _NW + 7) // 8 * 8
    n_ptr_pad = _NW * d_per + 24
    ptr_p = jnp.concatenate(
        [ptr, jnp.full((n_ptr_pad - (n_dst + 1),), e, dtype=ptr.dtype)])
    n_ad_pad = _NW * d_per + 16
    a_d = jnp.concatenate(
        [a_d, jnp.zeros((n_ad_pad - n_dst,), dtype=a_d.dtype)])
    e_pad = ((e + _BLKC * _CHUNK - 1) // (_BLKC * _CHUNK)) * (_BLKC * _CHUNK)
    idx_p = jnp.concatenate(
        [idx, jnp.zeros((e_pad - e,), dtype=idx.dtype)]
    ).reshape(e_pad // _CHUNK, _CHUNK)

    sc = _make_sc_kernel(n, n_dst, n_ptr_pad, n_ad_pad, hc, d_per)
    out_pad = sc(ptr_p, idx_p, xw, a_s, a_d, bias.astype(jnp.float32))
    return out_pad[:n_dst]


# speedup vs baseline: 1.3693x; 1.3693x over previous
import functools

import jax
import jax.numpy as jnp
from jax import lax
from jax.experimental import pallas as pl
from jax.experimental.pallas import tpu as pltpu
from jax.experimental.pallas import tpu_sc as plsc

NEG_SLOPE = 0.2
EPS = 1e-16

_NC = 2
_NS = 16
_NW = _NC * _NS
_L = 16
_CHUNK = 128


def _tc_body(x_ref, wt_ref, as_ref, ad_ref, xw_ref, asum_ref, adsum_ref):
    xw = jnp.dot(x_ref[...], wt_ref[...], preferred_element_type=jnp.float32)
    xw_ref[...] = xw
    asum_ref[...] = jnp.sum(xw * as_ref[...], axis=1, keepdims=True)
    adsum_ref[...] = jnp.sum(xw * ad_ref[...], axis=1, keepdims=True)


def _make_sc_kernel(n_src, n_dst, n_ptr_pad, n_ad_pad, c_dim, d_per):
    n_pad = _NW * d_per
    fvecs = c_dim // _L

    def body(ptr_hbm, idx_hbm, xw_hbm, as_hbm, ad_hbm, bias_hbm, out_hbm,
             ptr_v, as_v, ad_v, bias_v,
             blk_v, rows_a, rows_b,
             asg_v, s_v, out_v, den_v, sem_a, sem_b):
        cid = lax.axis_index("c")
        sid = lax.axis_index("s")
        wid = sid * _NC + cid
        d0 = wid * d_per
        d_hi = jnp.minimum(d0 + d_per, n_dst)

        pltpu.sync_copy(ptr_hbm.at[pl.ds(d0, d_per + 24)], ptr_v)
        pltpu.sync_copy(as_hbm, as_v)
        pltpu.sync_copy(ad_hbm.at[pl.ds(d0, d_per + 16)], ad_v)
        pltpu.sync_copy(bias_hbm, bias_v)

        zero16 = jnp.zeros((_L,), jnp.float32)

        def iload(ref, i):
            return ref[pl.ds(i, _L)][0]

        def zrow(r, _):
            for f in range(fvecs):
                out_v[r, pl.ds(f * _L, _L)] = zero16
            den_v[r, :] = zero16
            return 0
        lax.fori_loop(0, d_per, zrow, 0)

        e_lo = iload(ptr_v, 0)
        e_hi = iload(ptr_v, d_hi - d0)
        c_lo = e_lo // _CHUNK
        c_hi = (e_hi + _CHUNK - 1) // _CHUNK
        nchunks = c_hi - c_lo

        def ensure_block(c):
            @pl.when((c & 7) == 0)
            def _():
                b = c // 8
                pltpu.sync_copy(idx_hbm.at[pl.ds(b * 8, 8)],
                                blk_v.at[b & 1])

        def fetch(c, rows_buf, sem):
            pltpu.make_async_copy(
                xw_hbm.at[blk_v.at[(c // 8) & 1, c & 7]], rows_buf, sem
            ).start()

        def process_run(d, ce, rows_buf):
            dloc = d - d0
            es = iload(ptr_v, dloc)
            ee = iload(ptr_v, dloc + 1)
            run_lo = jnp.maximum(es, ce)
            run_hi = jnp.minimum(ee, ce + _CHUNK)

            @pl.when(run_hi > run_lo)
            def _():
                a_d_s = iload(ad_v, dloc)
                v_lo = (run_lo - ce) // _L
                v_hi = (run_hi - ce + _L - 1) // _L

                acc0 = tuple(out_v[dloc, pl.ds(f * _L, _L)]
                             for f in range(fvecs))

                def vec_body(vi, acc):
                    base = ce + vi * _L
                    pos = base + lax.iota(jnp.int32, _L)
                    m = (pos >= run_lo) & (pos < run_hi)
                    a = asg_v[pl.ds(vi * _L, _L)] + a_d_s
                    a = jnp.where(a >= 0, a, NEG_SLOPE * a)
                    s = jnp.where(m, jnp.exp(a), 0.0)
                    plsc.addupdate(den_v.at[dloc, :], s)
                    acc = list(acc)
                    for j in range(_L):
                        sj = s[j]
                        r = vi * _L + j
                        for f in range(fvecs):
                            acc[f] = acc[f] + sj * rows_buf[r, pl.ds(f * _L, _L)]
                    return tuple(acc)
                acc = lax.fori_loop(v_lo, v_hi, vec_body, acc0)
                for f in range(fvecs):
                    out_v[dloc, pl.ds(f * _L, _L)] = acc[f]

        def process(c, d, rows_cur, sem_cur, rows_nxt, sem_nxt):
            @pl.when(c + 1 < c_hi)
            def _():
                ensure_block(c + 1)
                fetch(c + 1, rows_nxt, sem_nxt)
            for v in range(_CHUNK // _L):
                ivec = blk_v[(c // 8) & 1, c & 7, pl.ds(v * _L, _L)]
                asg_v[pl.ds(v * _L, _L)] = plsc.load_gather(as_v, [ivec])
            pltpu.make_async_copy(
                xw_hbm.at[blk_v.at[(c // 8) & 1, c & 7]], rows_cur, sem_cur
            ).wait()
            ce = c * _CHUNK
            def wcond(dd):
                return jnp.logical_and(dd < d_hi,
                                       iload(ptr_v, dd + 1 - d0) <= ce + _CHUNK)

            def wbody(dd):
                process_run(dd, ce, rows_cur)
                return dd + 1
            d = lax.while_loop(wcond, wbody, d)

            @pl.when(jnp.logical_and(d < d_hi, iload(ptr_v, d - d0) < ce + _CHUNK))
            def _():
                process_run(d, ce, rows_cur)
            return d

        @pl.when(nchunks > 0)
        def _():
            b0 = c_lo // 8
            pltpu.sync_copy(idx_hbm.at[pl.ds(b0 * 8, 8)], blk_v.at[b0 & 1])
            fetch(c_lo, rows_a, sem_a)

        npairs = (nchunks + 1) // 2

        def pair_body(p, d):
            c0 = c_lo + 2 * p
            d = process(c0, d, rows_a, sem_a, rows_b, sem_b)
            d = lax.cond(
                c0 + 1 < c_hi,
                lambda dd: process(c0 + 1, dd, rows_b, sem_b,
                                   rows_a, sem_a),
                lambda dd: dd,
                d)
            return d
        lax.fori_loop(0, npairs, pair_body, d0)

        def fin(r, _):
            dtot = jnp.sum(den_v[r, :])
            denv = jnp.full((_L,), dtot) + EPS
            for f in range(fvecs):
                out_v[r, pl.ds(f * _L, _L)] = (
                    out_v[r, pl.ds(f * _L, _L)] / denv
                    + bias_v[pl.ds(f * _L, _L)])
            return 0
        lax.fori_loop(0, d_per, fin, 0)
        pltpu.sync_copy(out_v, out_hbm.at[pl.ds(d0, d_per)])

    mesh = plsc.VectorSubcoreMesh(core_axis_name="c", subcore_axis_name="s",
                                  num_cores=_NC, num_subcores=_NS)
    return pl.kernel(
        body,
        out_type=jax.ShapeDtypeStruct((n_pad, c_dim), jnp.float32),
        mesh=mesh,
        compiler_params=pltpu.CompilerParams(needs_layout_passes=False),
        scratch_types=[
            pltpu.VMEM((d_per + 24,), jnp.int32),
            pltpu.VMEM((n_src,), jnp.float32),
            pltpu.VMEM((d_per + 16,), jnp.float32),
            pltpu.VMEM((c_dim,), jnp.float32),
            pltpu.VMEM((2, 8, _CHUNK), jnp.int32),
            pltpu.VMEM((_CHUNK, c_dim), jnp.float32),
            pltpu.VMEM((_CHUNK, c_dim), jnp.float32),
            pltpu.VMEM((_CHUNK,), jnp.float32),
            pltpu.VMEM((2 * _L,), jnp.float32),
            pltpu.VMEM((d_per, c_dim), jnp.float32),
            pltpu.VMEM((d_per, _L), jnp.float32),
            pltpu.SemaphoreType.DMA,
            pltpu.SemaphoreType.DMA,
        ],
    )


def kernel(x, ptr, idx, num_dst, num_src, num_edge, lin_src, att_src, att_dst, bias):
    n, d_in = x.shape
    hc = lin_src.shape[0]
    n_dst = ptr.shape[0] - 1
    e = idx.shape[0]

    rb = 2000
    tc = pl.pallas_call(
        _tc_body,
        grid=(n // rb,),
        in_specs=[
            pl.BlockSpec((rb, d_in), lambda i: (i, 0)),
            pl.BlockSpec((d_in, hc), lambda i: (0, 0)),
            pl.BlockSpec((1, hc), lambda i: (0, 0)),
            pl.BlockSpec((1, hc), lambda i: (0, 0)),
        ],
        out_specs=[
            pl.BlockSpec((rb, hc), lambda i: (i, 0)),
            pl.BlockSpec((rb, 1), lambda i: (i, 0)),
            pl.BlockSpec((rb, 1), lambda i: (i, 0)),
        ],
        out_shape=[
            jax.ShapeDtypeStruct((n, hc), jnp.float32),
            jax.ShapeDtypeStruct((n, 1), jnp.float32),
            jax.ShapeDtypeStruct((n, 1), jnp.float32),
        ],
    )
    xw, a_s, a_d = tc(x, lin_src.T, att_src.reshape(1, hc), att_dst.reshape(1, hc))
    a_s = a_s.reshape(n)
    a_d = a_d.reshape(n)[:n_dst]

    d_per = ((n_dst + _NW - 1) // _NW + 7) // 8 * 8
    n_ptr_pad = _NW * d_per + 24
    ptr_p = jnp.concatenate(
        [ptr, jnp.full((n_ptr_pad - (n_dst + 1),), e, dtype=ptr.dtype)])
    n_ad_pad = _NW * d_per + 16
    a_d = jnp.concatenate(
        [a_d, jnp.zeros((n_ad_pad - n_dst,), dtype=a_d.dtype)])
    e_pad = ((e + 8 * _CHUNK - 1) // (8 * _CHUNK)) * (8 * _CHUNK)
    idx_p = jnp.concatenate(
        [idx, jnp.zeros((e_pad - e,), dtype=idx.dtype)]
    ).reshape(e_pad // _CHUNK, _CHUNK)

    sc = _make_sc_kernel(n, n_dst, n_ptr_pad, n_ad_pad, hc, d_per)
    out_pad = sc(ptr_p, idx_p, xw, a_s, a_d, bias.astype(jnp.float32))
    return out_pad[:n_dst]
